# baseline (speedup 1.0000x reference)
import functools

import jax
import jax.numpy as jnp
from jax import lax
from jax.experimental import pallas as pl
from jax.experimental.pallas import tpu as pltpu
from jax.experimental.pallas import tpu_sc as plsc

_B, _S, _D = 4, 4096, 2048
_NC, _NS = 2, 16
_NW = _NC * _NS
_DCHUNKS = _NW // _B
_DW = _D // _DCHUNKS
_L = 16
_JV = _DW // _L
_R = 32
_NCHUNK = _S // _R
_NBUF = 8
_PF = 5

_mesh = plsc.VectorSubcoreMesh(core_axis_name="c", subcore_axis_name="s")


@functools.partial(
    pl.kernel,
    out_type=jax.ShapeDtypeStruct((_B, _S, _D), jnp.float32),
    mesh=_mesh,
    scratch_types=[
        *[pltpu.VMEM((_R, _DW), jnp.float32) for _ in range(_NBUF)],
        *[pltpu.SemaphoreType.DMA for _ in range(2 * _NBUF)],
    ],
)
def _sc_cumsum(x_hbm, o_hbm, *scratch):
    bufs = scratch[:_NBUF]
    isems = scratch[_NBUF : 2 * _NBUF]
    osems = scratch[2 * _NBUF :]
    wid = lax.axis_index("s") * _NC + lax.axis_index("c")
    b = wid // _DCHUNKS
    d0 = (wid % _DCHUNKS) * _DW

    def load(g, k):
        return pltpu.make_async_copy(
            x_hbm.at[b, pl.ds(g * _R, _R), pl.ds(d0, _DW)],
            bufs[k],
            isems[k],
        )

    def store(g, k):
        return pltpu.make_async_copy(
            bufs[k],
            o_hbm.at[b, pl.ds(g * _R, _R), pl.ds(d0, _DW)],
            osems[k],
        )

    for g in range(_PF):
        load(g, g % _NBUF).start()

    zero = tuple(jnp.zeros((_L,), jnp.float32) for _ in range(_JV))

    def outer(i, carry):
        for k in range(_NBUF):
            g = i * _NBUF + k
            buf = bufs[k]

            @pl.when(g >= _NBUF - _PF)
            def _():
                store(g - (_NBUF - _PF), (k + _PF) % _NBUF).wait()

            @pl.when(g + _PF < _NCHUNK)
            def _():
                load(g + _PF, (k + _PF) % _NBUF).start()

            load(g, k).wait()

            def row_body(r, c):
                new = []
                for j in range(_JV):
                    cj = c[j] + buf[r, pl.ds(j * _L, _L)]
                    buf[r, pl.ds(j * _L, _L)] = cj
                    new.append(cj)
                return tuple(new)

            carry = lax.fori_loop(0, _R, row_body, carry)
            store(g, k).start()
        return carry

    lax.fori_loop(0, _NCHUNK // _NBUF, outer, zero)

    for g in range(_NCHUNK - (_NBUF - _PF), _NCHUNK):
        store(g, g % _NBUF).wait()


def kernel(input):
    return _sc_cumsum(input)


# speedup vs baseline: 1.0029x; 1.0029x over previous
import functools

import jax
import jax.numpy as jnp
from jax import lax
from jax.experimental import pallas as pl
from jax.experimental.pallas import tpu as pltpu
from jax.experimental.pallas import tpu_sc as plsc

_B, _S, _D = 4, 4096, 2048
_NC, _NS = 2, 16
_NW = _NC * _NS
_DCHUNKS = _NW // _B
_DW = _D // _DCHUNKS
_L = 16
_JV = _DW // _L
_R = 32
_NCHUNK = _S // _R
_NBUF = 8
_PF = 5

_mesh = plsc.VectorSubcoreMesh(core_axis_name="c", subcore_axis_name="s")


@functools.partial(
    pl.kernel,
    out_type=jax.ShapeDtypeStruct((_B, _S, _D), jnp.float32),
    mesh=_mesh,
    scratch_types=[
        *[pltpu.VMEM((_R, _DW), jnp.float32) for _ in range(_NBUF)],
        *[pltpu.SemaphoreType.DMA for _ in range(2 * _NBUF)],
    ],
)
def _sc_cumsum(x_hbm, o_hbm, *scratch):
    bufs = scratch[:_NBUF]
    isems = scratch[_NBUF : 2 * _NBUF]
    osems = scratch[2 * _NBUF :]
    wid = lax.axis_index("c") * _NS + lax.axis_index("s")
    b = wid // _DCHUNKS
    d0 = (wid % _DCHUNKS) * _DW

    def load(g, k):
        return pltpu.make_async_copy(
            x_hbm.at[b, pl.ds(g * _R, _R), pl.ds(d0, _DW)],
            bufs[k],
            isems[k],
        )

    def store(g, k):
        return pltpu.make_async_copy(
            bufs[k],
            o_hbm.at[b, pl.ds(g * _R, _R), pl.ds(d0, _DW)],
            osems[k],
        )

    for g in range(_PF):
        load(g, g % _NBUF).start()

    zero = tuple(jnp.zeros((_L,), jnp.float32) for _ in range(_JV))

    def outer(i, carry):
        for k in range(_NBUF):
            g = i * _NBUF + k
            buf = bufs[k]

            @pl.when(g >= _NBUF - _PF)
            def _():
                store(g - (_NBUF - _PF), (k + _PF) % _NBUF).wait()

            @pl.when(g + _PF < _NCHUNK)
            def _():
                load(g + _PF, (k + _PF) % _NBUF).start()

            load(g, k).wait()

            def row_body(r, c):
                new = []
                for j in range(_JV):
                    cj = c[j] + buf[r, pl.ds(j * _L, _L)]
                    buf[r, pl.ds(j * _L, _L)] = cj
                    new.append(cj)
                return tuple(new)

            carry = lax.fori_loop(0, _R, row_body, carry)
            store(g, k).start()
        return carry

    lax.fori_loop(0, _NCHUNK // _NBUF, outer, zero)

    for g in range(_NCHUNK - (_NBUF - _PF), _NCHUNK):
        store(g, g % _NBUF).wait()


def kernel(input):
    return _sc_cumsum(input)
